# baseline (speedup 1.0000x reference)
import jax
import jax.numpy as jnp
from jax.experimental import pallas as pl

_C = 256
_H = 32
_W = 32
_B = 2
_N = _B * _H * _W


def _shift(x, o):
    if o > 0:
        return jnp.concatenate(
            [x[:, o:], jnp.zeros((x.shape[0], o), jnp.float32)], axis=1)
    if o < 0:
        return jnp.concatenate(
            [jnp.zeros((x.shape[0], -o), jnp.float32), x[:, : _N + o]], axis=1)
    return x


def _conv3x3(x, w_taps, b, co):
    pix = jax.lax.broadcasted_iota(jnp.int32, (_C, _N), 1)
    wpos = pix % _W
    hpos = (pix // _W) % _H
    acc = jnp.zeros((co, _N), jnp.float32)
    for kh in range(3):
        for kw in range(3):
            o = (kh - 1) * _W + (kw - 1)
            xs = _shift(x, o)
            ws = wpos + (kw - 1)
            hs = hpos + (kh - 1)
            valid = (ws >= 0) & (ws < _W) & (hs >= 0) & (hs < _H)
            xs = jnp.where(valid, xs, 0.0)
            acc = acc + jnp.dot(w_taps[kh * 3 + kw], xs,
                                preferred_element_type=jnp.float32)
    return acc + b


def _tower(x, wh, bh, wo, bo, co):
    h = x
    for i in range(4):
        h = jnp.maximum(_conv3x3(h, wh[i], bh[i], _C), 0.0)
    return _conv3x3(h, wo, bo, co)


def _body(x_ref, cls_wh_ref, cls_bh_ref, cls_wo_ref, cls_bo_ref,
          box_wh_ref, box_bh_ref, box_wo_ref, box_bo_ref,
          cls_out_ref, box_out_ref):
    x = x_ref[...]
    cls_out_ref[...] = _tower(x, cls_wh_ref, cls_bh_ref,
                              cls_wo_ref[...], cls_bo_ref[...], 108)
    box_out_ref[...] = _tower(x, box_wh_ref, box_bh_ref,
                              box_wo_ref[...], box_bo_ref[...], 36)


def _taps(w):
    return jnp.transpose(w, (2, 3, 0, 1)).reshape(9, w.shape[0], w.shape[1])


def kernel(feats, cls_w, cls_b, cls_wo, cls_bo, box_w, box_b, box_wo, box_bo):
    x = jnp.transpose(feats[0], (1, 0, 2, 3)).reshape(_C, _N)
    cls_wh = jnp.stack([_taps(cls_w[i]) for i in range(4)])
    box_wh = jnp.stack([_taps(box_w[i]) for i in range(4)])
    cls_wot = _taps(cls_wo)
    box_wot = _taps(box_wo)
    cls_bh = cls_b[:, :, None]
    box_bh = box_b[:, :, None]
    cls_bot = cls_bo[:, None]
    box_bot = box_bo[:, None]

    cls_flat, box_flat = pl.pallas_call(
        _body,
        out_shape=(
            jax.ShapeDtypeStruct((108, _N), jnp.float32),
            jax.ShapeDtypeStruct((36, _N), jnp.float32),
        ),
    )(x, cls_wh, cls_bh, cls_wot, cls_bot,
      box_wh, box_bh, box_wot, box_bot)

    cls_out = jnp.transpose(cls_flat.reshape(108, _B, _H, _W), (1, 0, 2, 3))
    box_out = jnp.transpose(box_flat.reshape(36, _B, _H, _W), (1, 0, 2, 3))
    return (cls_out, box_out)


# speedup vs baseline: 1.3300x; 1.3300x over previous
import jax
import jax.numpy as jnp
from jax.experimental import pallas as pl

_C = 256
_H = 32
_W = 32
_B = 2
_N = _B * _H * _W


def _shift(x, o):
    if o > 0:
        return jnp.concatenate(
            [x[:, o:], jnp.zeros((x.shape[0], o), x.dtype)], axis=1)
    if o < 0:
        return jnp.concatenate(
            [jnp.zeros((x.shape[0], -o), x.dtype), x[:, : _N + o]], axis=1)
    return x


def _conv3x3(x, w_taps, b, co, masks):
    m_l, m_r, m_u, m_d = masks
    xs = (
        jnp.where(m_l, _shift(x, -1), 0),
        x,
        jnp.where(m_r, _shift(x, 1), 0),
    )
    acc = jnp.broadcast_to(b, (co, _N)).astype(jnp.float32)
    for kh in range(3):
        p = jnp.dot(w_taps[kh * 3], xs[0], preferred_element_type=jnp.float32)
        p = p + jnp.dot(w_taps[kh * 3 + 1], xs[1],
                        preferred_element_type=jnp.float32)
        p = p + jnp.dot(w_taps[kh * 3 + 2], xs[2],
                        preferred_element_type=jnp.float32)
        if kh == 0:
            acc = acc + jnp.where(m_u[:co], _shift(p, -_W), 0)
        elif kh == 2:
            acc = acc + jnp.where(m_d[:co], _shift(p, _W), 0)
        else:
            acc = acc + p
    return acc


def _tower(x, wh, bh, wo, bo, co, masks):
    h = x
    for i in range(4):
        a = _conv3x3(h, wh[i], bh[i], _C, masks)
        h = jnp.maximum(a, 0.0).astype(jnp.bfloat16)
    return _conv3x3(h, wo, bo, co, masks)


def _body(x_ref, cls_wh_ref, cls_bh_ref, cls_wo_ref, cls_bo_ref,
          box_wh_ref, box_bh_ref, box_wo_ref, box_bo_ref,
          cls_out_ref, box_out_ref):
    pix = jax.lax.broadcasted_iota(jnp.int32, (_C, _N), 1)
    wpos = pix % _W
    hpos = (pix // _W) % _H
    masks = (wpos >= 1, wpos <= _W - 2, hpos >= 1, hpos <= _H - 2)
    x = x_ref[...]
    cls_out_ref[...] = _tower(x, cls_wh_ref, cls_bh_ref,
                              cls_wo_ref[...], cls_bo_ref[...], 108, masks)
    box_out_ref[...] = _tower(x, box_wh_ref, box_bh_ref,
                              box_wo_ref[...], box_bo_ref[...], 36, masks)


def _taps(w):
    t = jnp.transpose(w, (2, 3, 0, 1)).reshape(9, w.shape[0], w.shape[1])
    return t.astype(jnp.bfloat16)


def kernel(feats, cls_w, cls_b, cls_wo, cls_bo, box_w, box_b, box_wo, box_bo):
    x = jnp.transpose(feats[0], (1, 0, 2, 3)).reshape(_C, _N)
    x = x.astype(jnp.bfloat16)
    cls_wh = jnp.stack([_taps(cls_w[i]) for i in range(4)])
    box_wh = jnp.stack([_taps(box_w[i]) for i in range(4)])
    cls_wot = _taps(cls_wo)
    box_wot = _taps(box_wo)
    cls_bh = cls_b[:, :, None]
    box_bh = box_b[:, :, None]
    cls_bot = cls_bo[:, None]
    box_bot = box_bo[:, None]

    cls_flat, box_flat = pl.pallas_call(
        _body,
        out_shape=(
            jax.ShapeDtypeStruct((108, _N), jnp.float32),
            jax.ShapeDtypeStruct((36, _N), jnp.float32),
        ),
    )(x, cls_wh, cls_bh, cls_wot, cls_bot,
      box_wh, box_bh, box_wot, box_bot)

    cls_out = jnp.transpose(cls_flat.reshape(108, _B, _H, _W), (1, 0, 2, 3))
    box_out = jnp.transpose(box_flat.reshape(36, _B, _H, _W), (1, 0, 2, 3))
    return (cls_out, box_out)
